# Initial kernel scaffold; baseline (speedup 1.0000x reference)
import functools

import jax
import jax.numpy as jnp
from jax import lax
from jax.experimental import pallas as pl
from jax.experimental.pallas import tpu as pltpu
from jax.experimental.pallas import tpu_sc as plsc

N_NODES = 10000
N_EDGES = 320000
FEAT = 128

NUM_CORES = 2
NUM_SUBCORES = 16
NUM_WORKERS = NUM_CORES * NUM_SUBCORES

CHUNK = 128
CHUNKS_PER_WORKER = -(-N_EDGES // (NUM_WORKERS * CHUNK))
EDGES_PER_WORKER = CHUNKS_PER_WORKER * CHUNK
E_PAD = NUM_WORKERS * EDGES_PER_WORKER

N_PAD = 10240
ROWS_PER_TILE = N_PAD // NUM_SUBCORES


def _sc_agg_body(with_cnt, h_hbm, src_hbm, dst_hbm, agg_hbm, cnt_hbm,
                 src_idx, dst_idx, rowbuf, ones_v, cnt_buf, sem):
    cid = lax.axis_index("c")
    sid = lax.axis_index("s")
    wid = cid * NUM_SUBCORES + sid

    def scoped(acc, cnt_acc):
        zero16 = jnp.zeros((16,), jnp.float32)

        @pl.loop(0, (CHUNK * FEAT) // 16)
        def _(i):
            rowbuf[pl.ds(i * 16, 16)] = zero16

        base = sid * ROWS_PER_TILE
        rows2d = rowbuf.reshape(CHUNK, FEAT)
        for k in range(ROWS_PER_TILE // CHUNK):
            pltpu.sync_copy(rows2d, acc.at[pl.ds(base + k * CHUNK, CHUNK)])
        if with_cnt:
            for j in range(CHUNK // 16):
                ones_v[pl.ds(j * 16, 16)] = jnp.ones((16,), jnp.float32)
            for k in range(ROWS_PER_TILE // CHUNK):
                pltpu.sync_copy(
                    rowbuf.at[pl.ds(0, CHUNK)],
                    cnt_acc.at[pl.ds(base + k * CHUNK, CHUNK)],
                )
        plsc.subcore_barrier()

        pltpu.sync_copy(src_hbm.at[wid], src_idx)
        pltpu.sync_copy(dst_hbm.at[wid], dst_idx)

        @pl.loop(0, CHUNKS_PER_WORKER)
        def _(c):
            gat = pltpu.async_copy(h_hbm.at[src_idx.at[c]], rows2d, sem)
            gat.wait()
            pltpu.sync_copy(rows2d, acc.at[dst_idx.at[c]], add=True)
            if with_cnt:
                pltpu.sync_copy(ones_v, cnt_acc.at[dst_idx.at[c]], add=True)

        plsc.subcore_barrier()

        for k in range(ROWS_PER_TILE // CHUNK):
            off = base + k * CHUNK
            pltpu.sync_copy(acc.at[pl.ds(off, CHUNK)], rows2d)
            pltpu.sync_copy(rows2d, agg_hbm.at[cid].at[pl.ds(off, CHUNK)])
        if with_cnt:
            pltpu.sync_copy(cnt_acc.at[pl.ds(base, ROWS_PER_TILE)], cnt_buf)
            pltpu.sync_copy(cnt_buf, cnt_hbm.at[cid].at[pl.ds(base, ROWS_PER_TILE)])

    pl.run_scoped(
        scoped,
        acc=pltpu.VMEM_SHARED((N_PAD, FEAT), jnp.float32),
        cnt_acc=pltpu.VMEM_SHARED((N_PAD,), jnp.float32),
    )


def _make_sc_agg(with_cnt):
    outs = [jax.ShapeDtypeStruct((NUM_CORES, N_PAD, FEAT), jnp.float32)]
    if with_cnt:
        outs.append(jax.ShapeDtypeStruct((NUM_CORES, N_PAD), jnp.float32))
    scratch = [
        pltpu.VMEM((CHUNKS_PER_WORKER, CHUNK), jnp.int32),
        pltpu.VMEM((CHUNKS_PER_WORKER, CHUNK), jnp.int32),
        pltpu.VMEM((CHUNK * FEAT,), jnp.float32),
        pltpu.VMEM((CHUNK,), jnp.float32),
        pltpu.VMEM((ROWS_PER_TILE,), jnp.float32),
        pltpu.SemaphoreType.DMA,
    ]
    body = functools.partial(_sc_agg_body, with_cnt)
    return pl.kernel(
        body,
        out_type=tuple(outs) if with_cnt else outs[0],
        mesh=plsc.VectorSubcoreMesh(core_axis_name="c", subcore_axis_name="s"),
        scratch_types=scratch,
        name="sage_sc_agg" + ("_cnt" if with_cnt else ""),
    )


def _tc_body(relu, agg_ref, cnt_ref, h_ref, wl_ref, wr_ref, b_ref, o_ref):
    agg = agg_ref[0, :N_NODES, :] + agg_ref[1, :N_NODES, :]
    cnt = cnt_ref[0, :N_NODES] + cnt_ref[1, :N_NODES]
    inv = 1.0 / jnp.maximum(cnt, 1.0)
    mean = agg * inv[:, None]
    out = (
        jnp.dot(mean, wl_ref[...].T, preferred_element_type=jnp.float32)
        + b_ref[...]
        + jnp.dot(h_ref[...], wr_ref[...].T, preferred_element_type=jnp.float32)
    )
    if relu:
        out = jnp.maximum(out, 0.0)
    o_ref[...] = out


def _make_tc(relu):
    return pl.pallas_call(
        functools.partial(_tc_body, relu),
        out_shape=jax.ShapeDtypeStruct((N_NODES, FEAT), jnp.float32),
        name="sage_tc_matmul",
    )


@jax.jit
def kernel(x, edge_index, W1l, b1, W1r, W2l, b2, W2r):
    src = edge_index[0]
    dst = edge_index[1]
    pad = E_PAD - N_EDGES
    src_p = jnp.concatenate([src, jnp.zeros((pad,), jnp.int32)])
    dst_p = jnp.concatenate([dst, jnp.full((pad,), N_NODES, jnp.int32)])
    src3 = src_p.reshape(NUM_WORKERS, CHUNKS_PER_WORKER, CHUNK)
    dst3 = dst_p.reshape(NUM_WORKERS, CHUNKS_PER_WORKER, CHUNK)

    agg1, cnt = _make_sc_agg(True)(x, src3, dst3)
    h = _make_tc(True)(agg1, cnt, x, W1l, W1r, b1)
    agg2 = _make_sc_agg(False)(h, src3, dst3)
    return _make_tc(False)(agg2, cnt, h, W2l, W2r, b2)


# speedup vs baseline: 5.0107x; 5.0107x over previous
import functools

import jax
import jax.numpy as jnp
from jax import lax
from jax.experimental import pallas as pl
from jax.experimental.pallas import tpu as pltpu
from jax.experimental.pallas import tpu_sc as plsc

N_NODES = 10000
N_EDGES = 320000
FEAT = 128

NUM_CORES = 2
NUM_SUBCORES = 16
NUM_WORKERS = NUM_CORES * NUM_SUBCORES

CHUNK = 128
CHUNKS_PER_WORKER = -(-N_EDGES // (NUM_WORKERS * CHUNK))
EDGES_PER_WORKER = CHUNKS_PER_WORKER * CHUNK
E_PAD = NUM_WORKERS * EDGES_PER_WORKER

N_PAD = 10240
ROWS_PER_TILE = N_PAD // NUM_SUBCORES


def _sc_agg_body(with_cnt, h_hbm, src_hbm, dst_hbm, agg_hbm, *rest):
    if with_cnt:
        cnt_hbm, src_idx, dst_idx, rowbuf, ones_v, cnt_buf, acc, cnt_acc, sem = rest
    else:
        cnt_hbm = None
        src_idx, dst_idx, rowbuf, ones_v, cnt_buf, acc, cnt_acc, sem = rest
    cid = lax.axis_index("c")
    sid = lax.axis_index("s")
    wid = cid * NUM_SUBCORES + sid

    zero16 = jnp.zeros((16,), jnp.float32)

    @pl.loop(0, CHUNK)
    def _(i):
        for j in range(FEAT // 16):
            rowbuf[i, pl.ds(j * 16, 16)] = zero16

    base = sid * ROWS_PER_TILE
    for k in range(ROWS_PER_TILE // CHUNK):
        pltpu.sync_copy(rowbuf, acc.at[pl.ds(base + k * CHUNK, CHUNK)])
    if with_cnt:
        for j in range(CHUNK // 16):
            ones_v[pl.ds(j * 16, 16)] = jnp.ones((16,), jnp.float32)
        for k in range(ROWS_PER_TILE // CHUNK):
            pltpu.sync_copy(
                rowbuf.at[0],
                cnt_acc.at[pl.ds(base + k * CHUNK, CHUNK)],
            )
    plsc.subcore_barrier()

    pltpu.sync_copy(src_hbm.at[wid], src_idx)
    pltpu.sync_copy(dst_hbm.at[wid], dst_idx)

    @pl.loop(0, CHUNKS_PER_WORKER)
    def _(c):
        gat = pltpu.async_copy(h_hbm.at[src_idx.at[c]], rowbuf, sem)
        gat.wait()
        pltpu.sync_copy(rowbuf, acc.at[dst_idx.at[c]], add=True)
        if with_cnt:
            pltpu.sync_copy(ones_v, cnt_acc.at[dst_idx.at[c]], add=True)

    plsc.subcore_barrier()

    for k in range(ROWS_PER_TILE // CHUNK):
        off = base + k * CHUNK
        pltpu.sync_copy(acc.at[pl.ds(off, CHUNK)], rowbuf)
        pltpu.sync_copy(rowbuf, agg_hbm.at[cid].at[pl.ds(off, CHUNK)])
    if with_cnt:
        pltpu.sync_copy(cnt_acc.at[pl.ds(base, ROWS_PER_TILE)], cnt_buf)
        pltpu.sync_copy(cnt_buf, cnt_hbm.at[cid].at[pl.ds(base, ROWS_PER_TILE)])


def _make_sc_agg(with_cnt):
    outs = [jax.ShapeDtypeStruct((NUM_CORES, N_PAD, FEAT), jnp.float32)]
    if with_cnt:
        outs.append(jax.ShapeDtypeStruct((NUM_CORES, N_PAD), jnp.float32))
    scratch = [
        pltpu.VMEM((CHUNKS_PER_WORKER, CHUNK), jnp.int32),
        pltpu.VMEM((CHUNKS_PER_WORKER, CHUNK), jnp.int32),
        pltpu.VMEM((CHUNK, FEAT), jnp.float32),
        pltpu.VMEM((CHUNK,), jnp.float32),
        pltpu.VMEM((ROWS_PER_TILE,), jnp.float32),
        pltpu.VMEM_SHARED((N_PAD, FEAT), jnp.float32),
        pltpu.VMEM_SHARED((N_PAD,), jnp.float32),
        pltpu.SemaphoreType.DMA,
    ]
    body = functools.partial(_sc_agg_body, with_cnt)
    return pl.kernel(
        body,
        out_type=tuple(outs) if with_cnt else outs[0],
        mesh=plsc.VectorSubcoreMesh(core_axis_name="c", subcore_axis_name="s"),
        scratch_types=scratch,
        name="sage_sc_agg" + ("_cnt" if with_cnt else ""),
    )


def _tc_body(relu, agg_ref, cnt_ref, h_ref, wl_ref, wr_ref, b_ref, o_ref):
    agg = agg_ref[0, :N_NODES, :] + agg_ref[1, :N_NODES, :]
    cnt = cnt_ref[0, :N_NODES] + cnt_ref[1, :N_NODES]
    inv = 1.0 / jnp.maximum(cnt, 1.0)
    mean = agg * inv[:, None]
    out = (
        jnp.dot(mean, wl_ref[...].T, preferred_element_type=jnp.float32)
        + b_ref[...]
        + jnp.dot(h_ref[...], wr_ref[...].T, preferred_element_type=jnp.float32)
    )
    if relu:
        out = jnp.maximum(out, 0.0)
    o_ref[...] = out


def _make_tc(relu):
    return pl.pallas_call(
        functools.partial(_tc_body, relu),
        out_shape=jax.ShapeDtypeStruct((N_NODES, FEAT), jnp.float32),
        name="sage_tc_matmul",
    )


@jax.jit
def kernel(x, edge_index, W1l, b1, W1r, W2l, b2, W2r):
    src = edge_index[0]
    dst = edge_index[1]
    pad = E_PAD - N_EDGES
    src_p = jnp.concatenate([src, jnp.zeros((pad,), jnp.int32)])
    dst_p = jnp.concatenate([dst, jnp.full((pad,), N_NODES, jnp.int32)])
    src3 = src_p.reshape(NUM_WORKERS, CHUNKS_PER_WORKER, CHUNK)
    dst3 = dst_p.reshape(NUM_WORKERS, CHUNKS_PER_WORKER, CHUNK)

    agg1, cnt = _make_sc_agg(True)(x, src3, dst3)
    h = _make_tc(True)(agg1, cnt, x, W1l, W1r, b1)
    agg2 = _make_sc_agg(False)(h, src3, dst3)
    return _make_tc(False)(agg2, cnt, h, W2l, W2r, b2)
